# baseline (speedup 1.0000x reference)
import functools

import jax
import jax.numpy as jnp
from jax import lax
from jax.experimental import pallas as pl
from jax.experimental.pallas import tpu as pltpu
from jax.experimental.pallas import tpu_sc as plsc

VOCAB = 1000000
EMBED_DIM = 64
BATCH = 16384


@functools.lru_cache(maxsize=None)
def _build_gather_kernel():
    info = plsc.get_sparse_core_info()
    nw = info.num_cores * info.num_subcores
    b_per_w = BATCH // nw
    mesh = plsc.VectorSubcoreMesh(core_axis_name="c", subcore_axis_name="s")

    @functools.partial(
        pl.kernel,
        mesh=mesh,
        out_type=jax.ShapeDtypeStruct((BATCH, EMBED_DIM), jnp.float32),
        scratch_types=[
            pltpu.VMEM((b_per_w,), jnp.int32),
            pltpu.VMEM((b_per_w, EMBED_DIM), jnp.float32),
            pltpu.SemaphoreType.DMA,
        ],
        compiler_params=pltpu.CompilerParams(use_tc_tiling_on_sc=False),
    )
    def gather(idx_hbm, table_hbm, out_hbm, idx_v, rows_v, sem):
        wid = lax.axis_index("s") * info.num_cores + lax.axis_index("c")
        base = wid * b_per_w
        pltpu.sync_copy(idx_hbm.at[pl.ds(base, b_per_w)], idx_v)
        pltpu.async_copy(table_hbm.at[idx_v], rows_v, sem).wait()
        pltpu.sync_copy(rows_v, out_hbm.at[pl.ds(base, b_per_w)])

    return gather


def kernel(target_w_id, L):
    gather = _build_gather_kernel()
    return gather(target_w_id.astype(jnp.int32), L)


# speedup vs baseline: 2.5254x; 2.5254x over previous
import functools

import jax
import jax.numpy as jnp
from jax import lax
from jax.experimental import pallas as pl
from jax.experimental.pallas import tpu as pltpu
from jax.experimental.pallas import tpu_sc as plsc

VOCAB = 1000000
EMBED_DIM = 64
BATCH = 16384

_TAIL_LO = 999936
_SLAB_TC = 248
_SLAB_STRIDE_TC = 244
_SLAB_COLS = _SLAB_TC * 128
_BLK = 256
_NBLK = _SLAB_COLS // _BLK
_STAGE_ROWS = 256
_OUT_ROWS = BATCH + _STAGE_ROWS
_BIG = 2**30


@functools.lru_cache(maxsize=None)
def _build_gather_kernel():
    info = plsc.get_sparse_core_info()
    nc = info.num_cores
    mesh = plsc.VectorSubcoreMesh(core_axis_name="c", subcore_axis_name="s")

    @functools.partial(
        pl.kernel,
        mesh=mesh,
        out_type=jax.ShapeDtypeStruct((_OUT_ROWS, 128), jnp.float32),
        scratch_types=[
            pltpu.VMEM((BATCH,), jnp.int32),
            pltpu.VMEM((BATCH,), jnp.int32),
            pltpu.VMEM((2, 64, _BLK), jnp.float32),
            pltpu.VMEM((_STAGE_ROWS, 128), jnp.float32),
            pltpu.VMEM((BATCH + 16,), jnp.int32),
            pltpu.VMEM((64, 64), jnp.float32),
            pltpu.VMEM((2, 128), jnp.int32),
            pltpu.SemaphoreType.DMA,
            pltpu.SemaphoreType.DMA,
        ],
        compiler_params=pltpu.CompilerParams(needs_layout_passes=False),
    )
    def gather(
        idx_hbm,
        lt_hbm,
        tail_hbm,
        out2_hbm,
        idx_all,
        jbuf,
        blockbuf,
        stage,
        lb,
        tailbuf,
        jchunk,
        sem_blk,
        sem_sc,
    ):
        iota16 = lax.iota(jnp.int32, 16)
        wid = lax.axis_index("s") * nc + lax.axis_index("c")
        c_lo = wid * (_SLAB_STRIDE_TC * 128)
        one_v = jnp.full((16,), 1, jnp.int32)

        def reinit_chunk(c):
            cv = jnp.full((16,), c, jnp.int32)
            for g in range(8):
                icv = jnp.full((16,), g * 16, jnp.int32) + iota16
                dummy = (
                    jnp.full((16,), BATCH + g * 16, jnp.int32)
                    + cv * 128
                    + iota16
                )
                plsc.store_scatter(jchunk, [cv, icv], dummy)

        reinit_chunk(jnp.int32(0))
        reinit_chunk(jnp.int32(1))

        pltpu.sync_copy(idx_hbm, idx_all)

        tail_lo = jnp.where(wid == 0, jnp.int32(_TAIL_LO), jnp.int32(_BIG))
        lo_v = jnp.full((16,), c_lo, jnp.int32)
        hi_v = jnp.full((16,), c_lo + _SLAB_COLS, jnp.int32)
        tail_v = jnp.full((16,), tail_lo, jnp.int32)

        def scan_body(g, cnt):
            iv = idx_all[pl.ds(g * 16, 16)]
            jv = jnp.full((16,), g * 16, jnp.int32) + iota16
            m = ((iv >= lo_v) & (iv < hi_v)) | (iv >= tail_v)
            pm = plsc.cumsum(m.astype(jnp.int32))
            tgt = jnp.full((16,), cnt, jnp.int32) + pm - one_v
            plsc.store_scatter(jbuf, [tgt], jv, mask=m)
            return cnt + pm[15]

        cnt = lax.fori_loop(0, BATCH // 16, scan_body, jnp.int32(0))
        n_groups = (cnt + 15) // 16
        cnt_v = jnp.full((16,), cnt, jnp.int32)

        def flush(chunk):
            off = pl.multiple_of(chunk * 128, 128)
            pltpu.async_copy(
                stage.at[pl.ds(off, 128), :],
                out2_hbm.at[jchunk.at[chunk]],
                sem_sc,
            ).wait()
            reinit_chunk(chunk)

        def process_block(blk_start, buf_ref, blk_w, scnt):
            blk_lo_v = jnp.full((16,), blk_start, jnp.int32)
            blk_hi_v = jnp.full((16,), blk_start + blk_w, jnp.int32)

            def cscan(g, nb):
                jv = jbuf[pl.ds(g * 16, 16)]
                pos = jnp.full((16,), g * 16, jnp.int32) + iota16
                valid = pos < cnt_v
                cols = plsc.load_gather(idx_all, [jv], mask=valid)
                lm = valid & (cols >= blk_lo_v) & (cols < blk_hi_v)
                pm = plsc.cumsum(lm.astype(jnp.int32))
                tgt = jnp.full((16,), nb, jnp.int32) + pm - one_v
                plsc.store_scatter(lb, [tgt], jv, mask=lm)
                return nb + pm[15]

            nb = lax.fori_loop(0, n_groups, cscan, jnp.int32(0))

            @pl.when(nb > 0)
            def _():
                j0 = lb[pl.ds(0, 16)][0]
                lb[pl.ds(nb, 16)] = jnp.full((16,), j0, jnp.int32)

            rowq = [
                jnp.full((16,), q * 16, jnp.int32) + iota16 for q in range(4)
            ]

            def ext(g, scnt_):
                jv = lb[pl.ds(g * 16, 16)]
                colv = plsc.load_gather(idx_all, [jv]) - blk_lo_v
                slots = (
                    jnp.full((16,), scnt_, jnp.int32) + iota16
                ) & jnp.full((16,), _STAGE_ROWS - 1, jnp.int32)
                for k in range(16):
                    cbv = jnp.full((16,), colv[k], jnp.int32)
                    sbv = jnp.full((16,), slots[k], jnp.int32)
                    for q in range(4):
                        vals = plsc.load_gather(buf_ref, [rowq[q], cbv])
                        plsc.store_scatter(stage, [sbv, rowq[q]], vals)
                chunk_v = lax.shift_right_logical(
                    slots, jnp.full((16,), 7, jnp.int32)
                )
                in_chunk_v = slots & jnp.full((16,), 127, jnp.int32)
                plsc.store_scatter(jchunk, [chunk_v, in_chunk_v], jv)
                new = scnt_ + 16

                @pl.when(new & 127 == 0)
                def _():
                    flush(((new - 1) >> 7) & 1)

                return new

            return lax.fori_loop(0, (nb + 15) // 16, ext, scnt)

        pltpu.sync_copy(lt_hbm.at[:, pl.ds(c_lo, _BLK)], blockbuf.at[0])

        def outer(t, scnt):
            for par in (0, 1):
                b = t * 2 + par
                nxt = jnp.minimum(b + 1, _NBLK - 1)
                nxt_off = pl.multiple_of(c_lo + nxt * _BLK, 128)
                pltpu.async_copy(
                    lt_hbm.at[:, pl.ds(nxt_off, _BLK)],
                    blockbuf.at[(par + 1) % 2],
                    sem_blk,
                )
                scnt = process_block(
                    c_lo + b * _BLK, blockbuf.at[par], _BLK, scnt
                )
                pltpu.make_async_copy(
                    lt_hbm.at[:, pl.ds(nxt_off, _BLK)],
                    blockbuf.at[(par + 1) % 2],
                    sem_blk,
                ).wait()
            return scnt

        scnt = lax.fori_loop(0, _NBLK // 2, outer, jnp.int32(0))

        pltpu.sync_copy(tail_hbm, tailbuf)
        scnt = process_block(jnp.int32(_TAIL_LO), tailbuf, 64, scnt)

        flush(jnp.int32(0))
        flush(jnp.int32(1))

    return gather


def kernel(target_w_id, L):
    gather = _build_gather_kernel()
    idx = target_w_id.astype(jnp.int32)
    tail_t = lax.slice(L, (_TAIL_LO, 0), (VOCAB, EMBED_DIM)).T
    out2 = gather(idx, L.T, tail_t)
    return out2[:BATCH, :EMBED_DIM]


# speedup vs baseline: 3.4305x; 1.3584x over previous
import functools

import jax
import jax.numpy as jnp
from jax import lax
from jax.experimental import pallas as pl
from jax.experimental.pallas import tpu as pltpu
from jax.experimental.pallas import tpu_sc as plsc

VOCAB = 1000000
EMBED_DIM = 64
BATCH = 16384

_TAIL_LO = 999936
_SLAB_TC = 248
_SLAB_STRIDE_TC = 244
_SLAB_COLS = _SLAB_TC * 128
_BLK = 512
_NBLK = _SLAB_COLS // _BLK
_STAGE_ROWS = 128
_NCHUNK = _STAGE_ROWS // 128
_OUT_ROWS = BATCH + _STAGE_ROWS
_BIG = 2**30


@functools.lru_cache(maxsize=None)
def _build_gather_kernel():
    info = plsc.get_sparse_core_info()
    nc = info.num_cores
    mesh = plsc.VectorSubcoreMesh(core_axis_name="c", subcore_axis_name="s")

    @functools.partial(
        pl.kernel,
        mesh=mesh,
        out_type=jax.ShapeDtypeStruct((_OUT_ROWS, 128), jnp.float32),
        scratch_types=[
            pltpu.VMEM((BATCH,), jnp.int32),
            pltpu.VMEM((BATCH,), jnp.int32),
            pltpu.VMEM((2, 64, _BLK), jnp.float32),
            pltpu.VMEM((_STAGE_ROWS, 128), jnp.float32),
            pltpu.VMEM((6160,), jnp.int32),
            pltpu.VMEM((64, 64), jnp.float32),
            pltpu.VMEM((_NCHUNK, 128), jnp.int32),
            pltpu.SemaphoreType.DMA,
            pltpu.SemaphoreType.DMA,
        ],
        compiler_params=pltpu.CompilerParams(needs_layout_passes=False),
    )
    def gather(
        idx_hbm,
        lt_hbm,
        tail_hbm,
        out2_hbm,
        idx_all,
        jbuf,
        blockbuf,
        stage,
        lb,
        tailbuf,
        jchunk,
        sem_blk,
        sem_sc,
    ):
        iota16 = lax.iota(jnp.int32, 16)
        wid = lax.axis_index("s") * nc + lax.axis_index("c")
        c_lo = wid * (_SLAB_STRIDE_TC * 128)
        one_v = jnp.full((16,), 1, jnp.int32)

        def reinit_chunk(c):
            cv = jnp.full((16,), c, jnp.int32)
            for g in range(8):
                icv = jnp.full((16,), g * 16, jnp.int32) + iota16
                dummy = (
                    jnp.full((16,), BATCH + g * 16, jnp.int32)
                    + cv * 128
                    + iota16
                )
                plsc.store_scatter(jchunk, [cv, icv], dummy)

        for c in range(_NCHUNK):
            reinit_chunk(jnp.int32(c))

        pltpu.sync_copy(idx_hbm, idx_all)

        tail_lo = jnp.where(wid == 0, jnp.int32(_TAIL_LO), jnp.int32(_BIG))
        lo_v = jnp.full((16,), c_lo, jnp.int32)
        hi_v = jnp.full((16,), c_lo + _SLAB_COLS, jnp.int32)
        tail_v = jnp.full((16,), tail_lo, jnp.int32)

        def scan_body(g, cnt):
            iv = idx_all[pl.ds(g * 16, 16)]
            jv = jnp.full((16,), g * 16, jnp.int32) + iota16
            m = ((iv >= lo_v) & (iv < hi_v)) | (iv >= tail_v)
            pm = plsc.cumsum(m.astype(jnp.int32))
            tgt = jnp.full((16,), cnt, jnp.int32) + pm - one_v
            plsc.store_scatter(jbuf, [tgt], jv, mask=m)
            return cnt + pm[15]

        cnt = lax.fori_loop(0, BATCH // 16, scan_body, jnp.int32(0))
        n_groups = (cnt + 15) // 16
        cnt_v = jnp.full((16,), cnt, jnp.int32)

        def flush(chunk):
            off = pl.multiple_of(chunk * 128, 128)
            pltpu.async_copy(
                stage.at[pl.ds(off, 128), :],
                out2_hbm.at[jchunk.at[chunk]],
                sem_sc,
            ).wait()
            reinit_chunk(chunk)

        def process_block(blk_start, buf_ref, blk_w, scnt):
            blk_lo_v = jnp.full((16,), blk_start, jnp.int32)
            blk_hi_v = jnp.full((16,), blk_start + blk_w, jnp.int32)

            def cscan(g, nb):
                jv = jbuf[pl.ds(g * 16, 16)]
                pos = jnp.full((16,), g * 16, jnp.int32) + iota16
                valid = pos < cnt_v
                cols = plsc.load_gather(idx_all, [jv], mask=valid)
                lm = valid & (cols >= blk_lo_v) & (cols < blk_hi_v)
                pm = plsc.cumsum(lm.astype(jnp.int32))
                tgt = jnp.full((16,), nb, jnp.int32) + pm - one_v
                plsc.store_scatter(lb, [tgt], jv, mask=lm)
                return nb + pm[15]

            nb = lax.fori_loop(0, n_groups, cscan, jnp.int32(0))

            @pl.when(nb > 0)
            def _():
                j0 = lb[pl.ds(0, 16)][0]
                lb[pl.ds(nb, 16)] = jnp.full((16,), j0, jnp.int32)

            rowq = [
                jnp.full((16,), q * 16, jnp.int32) + iota16 for q in range(4)
            ]

            def ext(g, scnt_):
                jv = lb[pl.ds(g * 16, 16)]
                colv = plsc.load_gather(idx_all, [jv]) - blk_lo_v
                slots = (
                    jnp.full((16,), scnt_, jnp.int32) + iota16
                ) & jnp.full((16,), _STAGE_ROWS - 1, jnp.int32)
                for k in range(16):
                    cbv = jnp.full((16,), colv[k], jnp.int32)
                    sbv = jnp.full((16,), slots[k], jnp.int32)
                    for q in range(4):
                        vals = plsc.load_gather(buf_ref, [rowq[q], cbv])
                        plsc.store_scatter(stage, [sbv, rowq[q]], vals)
                chunk_v = lax.shift_right_logical(
                    slots, jnp.full((16,), 7, jnp.int32)
                )
                in_chunk_v = slots & jnp.full((16,), 127, jnp.int32)
                plsc.store_scatter(jchunk, [chunk_v, in_chunk_v], jv)
                new = scnt_ + 16

                @pl.when(new & 127 == 0)
                def _():
                    flush(((new - 1) >> 7) & (_NCHUNK - 1))

                return new

            return lax.fori_loop(0, (nb + 15) // 16, ext, scnt)

        pltpu.sync_copy(lt_hbm.at[:, pl.ds(c_lo, _BLK)], blockbuf.at[0])

        def outer(t, scnt):
            for par in (0, 1):
                b = t * 2 + par
                nxt = jnp.minimum(b + 1, _NBLK - 1)
                nxt_off = pl.multiple_of(c_lo + nxt * _BLK, 128)
                pltpu.async_copy(
                    lt_hbm.at[:, pl.ds(nxt_off, _BLK)],
                    blockbuf.at[(par + 1) % 2],
                    sem_blk,
                )
                scnt = process_block(
                    c_lo + b * _BLK, blockbuf.at[par], _BLK, scnt
                )
                pltpu.make_async_copy(
                    lt_hbm.at[:, pl.ds(nxt_off, _BLK)],
                    blockbuf.at[(par + 1) % 2],
                    sem_blk,
                ).wait()
            return scnt

        scnt = lax.fori_loop(0, _NBLK // 2, outer, jnp.int32(0))

        pltpu.sync_copy(tail_hbm, tailbuf)
        scnt = process_block(jnp.int32(_TAIL_LO), tailbuf, 64, scnt)

        for c in range(_NCHUNK):
            flush(jnp.int32(c))

    return gather


def kernel(target_w_id, L):
    gather = _build_gather_kernel()
    idx = target_w_id.astype(jnp.int32)
    tail_t = lax.slice(L, (_TAIL_LO, 0), (VOCAB, EMBED_DIM)).T
    out2 = gather(idx, L.T, tail_t)
    return out2[:BATCH, :EMBED_DIM]
